# Initial kernel scaffold; baseline (speedup 1.0000x reference)
import jax
import jax.numpy as jnp
from jax.experimental import pallas as pl


def kernel(vert_features, edges, W_self, W_neigh, b):
    raise NotImplementedError("write your pallas kernel here")


# speedup vs baseline: 8.7978x; 8.7978x over previous
import functools

import jax
import jax.numpy as jnp
from jax import lax
from jax.experimental import pallas as pl
from jax.experimental.pallas import tpu as pltpu
from jax.experimental.pallas import tpu_sc as plsc

N_NODES = 10000
N_PAD = 10240
D = 128
N_EDGES = 320000
NW = 32
CHUNK = 80
NCHUNK = N_EDGES // NW // CHUNK
ROWS_PER_TILE = N_PAD // 16


def _sc_body(x_hbm, src_hbm, dst_hbm, s_out, deg_out,
             src_v, dst_v, rows_v, ones_v, zdeg_v, gsem, agg_sh, deg_sh):
    c = lax.axis_index("c")
    s = lax.axis_index("s")
    wid = c * 16 + s

    @pl.loop(0, CHUNK)
    def _(r):
        for j in range(D // 16):
            rows_v[r, pl.ds(j * 16, 16)] = jnp.zeros((16,), jnp.float32)

    @pl.loop(0, ROWS_PER_TILE // 16)
    def _(k):
        zdeg_v[pl.ds(k * 16, 16)] = jnp.zeros((16,), jnp.float32)

    for k in range(CHUNK // 16):
        ones_v[pl.ds(k * 16, 16)] = jnp.ones((16,), jnp.float32)

    for k in range(ROWS_PER_TILE // CHUNK):
        pltpu.sync_copy(rows_v, agg_sh.at[pl.ds(s * ROWS_PER_TILE + k * CHUNK, CHUNK)])
    pltpu.sync_copy(zdeg_v, deg_sh.at[pl.ds(s * ROWS_PER_TILE, ROWS_PER_TILE)])
    plsc.subcore_barrier()

    pltpu.sync_copy(src_hbm.at[wid], src_v)
    pltpu.sync_copy(dst_hbm.at[wid], dst_v)

    @pl.loop(0, NCHUNK)
    def _(ci):
        pltpu.async_copy(x_hbm.at[src_v.at[ci]], rows_v, gsem).wait()
        pltpu.sync_copy(rows_v, agg_sh.at[dst_v.at[ci]], add=True)
        pltpu.sync_copy(ones_v, deg_sh.at[dst_v.at[ci]], add=True)

    plsc.subcore_barrier()

    pltpu.sync_copy(agg_sh.at[pl.ds(s * ROWS_PER_TILE, ROWS_PER_TILE)],
                    s_out.at[c, pl.ds(s * ROWS_PER_TILE, ROWS_PER_TILE)])
    pltpu.sync_copy(deg_sh.at[pl.ds(s * ROWS_PER_TILE, ROWS_PER_TILE)],
                    deg_out.at[c, pl.ds(s * ROWS_PER_TILE, ROWS_PER_TILE)])


@jax.jit
def _sc_scatter(x, src, dst):
    mesh = plsc.VectorSubcoreMesh(core_axis_name="c", subcore_axis_name="s")
    return pl.kernel(
        _sc_body,
        out_type=(
            jax.ShapeDtypeStruct((2, N_PAD, D), jnp.float32),
            jax.ShapeDtypeStruct((2, N_PAD), jnp.float32),
        ),
        mesh=mesh,
        scratch_types=(
            pltpu.VMEM((NCHUNK, CHUNK), jnp.int32),
            pltpu.VMEM((NCHUNK, CHUNK), jnp.int32),
            pltpu.VMEM((CHUNK, D), jnp.float32),
            pltpu.VMEM((CHUNK,), jnp.float32),
            pltpu.VMEM((ROWS_PER_TILE,), jnp.float32),
            pltpu.SemaphoreType.DMA,
            pltpu.VMEM_SHARED((N_PAD, D), jnp.float32),
            pltpu.VMEM_SHARED((N_PAD,), jnp.float32),
        ),
    )(x, src, dst)


BM = 2048


def _tc_body(x_ref, s_ref, deg_ref, ws_ref, wn_ref, b_ref, o_ref):
    dg = jnp.maximum(deg_ref[0, :] + deg_ref[1, :], 1.0)
    a = (s_ref[0] + s_ref[1]) / dg[:, None]
    o_ref[...] = (
        jnp.dot(x_ref[...], ws_ref[...], preferred_element_type=jnp.float32)
        + jnp.dot(a, wn_ref[...], preferred_element_type=jnp.float32)
        + b_ref[...]
    )


@jax.jit
def _tc_combine(x, s_part, deg_part, w_self, w_neigh, b2d):
    grid = (N_PAD // BM,)
    return pl.pallas_call(
        _tc_body,
        grid=grid,
        in_specs=[
            pl.BlockSpec((BM, D), lambda i: (i, 0)),
            pl.BlockSpec((2, BM, D), lambda i: (0, i, 0)),
            pl.BlockSpec((2, BM), lambda i: (0, i)),
            pl.BlockSpec((D, D), lambda i: (0, 0)),
            pl.BlockSpec((D, D), lambda i: (0, 0)),
            pl.BlockSpec((1, D), lambda i: (0, 0)),
        ],
        out_specs=pl.BlockSpec((BM, D), lambda i: (i, 0)),
        out_shape=jax.ShapeDtypeStruct((N_NODES, D), jnp.float32),
    )(x, s_part, deg_part, w_self, w_neigh, b2d)


def kernel(vert_features, edges, W_self, W_neigh, b):
    x = vert_features
    src = edges[0].astype(jnp.int32).reshape(NW, NCHUNK, CHUNK)
    dst = edges[1].astype(jnp.int32).reshape(NW, NCHUNK, CHUNK)
    s_part, deg_part = _sc_scatter(x, src, dst)
    return _tc_combine(x, s_part, deg_part, W_self, W_neigh, b.reshape(1, D))


# speedup vs baseline: 9.1997x; 1.0457x over previous
import jax
import jax.numpy as jnp
from jax import lax
from jax.experimental import pallas as pl
from jax.experimental.pallas import tpu as pltpu
from jax.experimental.pallas import tpu_sc as plsc

N_NODES = 10000
N_PAD = 10240
D = 128
DH = D // 2
N_EDGES = 320000
NT = 16
CHUNK = 125
NCHUNK = N_EDGES // NT // CHUNK
ROWS_PER_TILE = N_PAD // NT


def _sc_body(xl_hbm, xr_hbm, src_hbm, dst_hbm, s_out, deg_out,
             src_v, dst_v, rows0_v, rows1_v, ones_v, zdeg_v, zrow_v,
             gsem0, gsem1, agg_sh, deg_sh):
    c = lax.axis_index("c")
    s = lax.axis_index("s")

    @pl.loop(0, 128)
    def _(r):
        for j in range(DH // 16):
            zrow_v[r, pl.ds(j * 16, 16)] = jnp.zeros((16,), jnp.float32)

    @pl.loop(0, ROWS_PER_TILE // 16)
    def _(k):
        zdeg_v[pl.ds(k * 16, 16)] = jnp.zeros((16,), jnp.float32)

    for k in range(8):
        ones_v[pl.ds(k * 16, 16)] = jnp.ones((16,), jnp.float32)

    for k in range(ROWS_PER_TILE // 128):
        pltpu.sync_copy(zrow_v, agg_sh.at[pl.ds(s * ROWS_PER_TILE + k * 128, 128)])
    pltpu.sync_copy(zdeg_v, deg_sh.at[pl.ds(s * ROWS_PER_TILE, ROWS_PER_TILE)])
    plsc.subcore_barrier()

    pltpu.sync_copy(src_hbm.at[s], src_v)
    pltpu.sync_copy(dst_hbm.at[s], dst_v)

    def main_loop(x_hbm, deg_first_half):
        @pl.loop(0, NCHUNK // 2)
        def _(p):
            c0 = 2 * p
            dcond = (p < NCHUNK // 4) if deg_first_half else (p >= NCHUNK // 4)
            g0 = pltpu.async_copy(x_hbm.at[src_v.at[c0]], rows0_v, gsem0)
            g1 = pltpu.async_copy(x_hbm.at[src_v.at[c0 + 1]], rows1_v, gsem1)
            g0.wait()

            @pl.when(dcond)
            def _():
                pltpu.sync_copy(ones_v.at[pl.ds(0, CHUNK)],
                                deg_sh.at[dst_v.at[c0]], add=True)

            pltpu.sync_copy(rows0_v, agg_sh.at[dst_v.at[c0]], add=True)
            g1.wait()

            @pl.when(dcond)
            def _():
                pltpu.sync_copy(ones_v.at[pl.ds(0, CHUNK)],
                                deg_sh.at[dst_v.at[c0 + 1]], add=True)

            pltpu.sync_copy(rows1_v, agg_sh.at[dst_v.at[c0 + 1]], add=True)

    @pl.when(c == 0)
    def _():
        main_loop(xl_hbm, True)

    @pl.when(c == 1)
    def _():
        main_loop(xr_hbm, False)

    plsc.subcore_barrier()

    pltpu.sync_copy(agg_sh.at[pl.ds(s * ROWS_PER_TILE, ROWS_PER_TILE)],
                    s_out.at[c, pl.ds(s * ROWS_PER_TILE, ROWS_PER_TILE)])
    pltpu.sync_copy(deg_sh.at[pl.ds(s * ROWS_PER_TILE, ROWS_PER_TILE)],
                    deg_out.at[c, pl.ds(s * ROWS_PER_TILE, ROWS_PER_TILE)])


@jax.jit
def _sc_scatter(xl, xr, src, dst):
    mesh = plsc.VectorSubcoreMesh(core_axis_name="c", subcore_axis_name="s")
    return pl.kernel(
        _sc_body,
        out_type=(
            jax.ShapeDtypeStruct((2, N_PAD, DH), jnp.float32),
            jax.ShapeDtypeStruct((2, N_PAD), jnp.float32),
        ),
        mesh=mesh,
        compiler_params=pltpu.CompilerParams(use_tc_tiling_on_sc=False),
        scratch_types=(
            pltpu.VMEM((NCHUNK, CHUNK), jnp.int32),
            pltpu.VMEM((NCHUNK, CHUNK), jnp.int32),
            pltpu.VMEM((CHUNK, DH), jnp.float32),
            pltpu.VMEM((CHUNK, DH), jnp.float32),
            pltpu.VMEM((128,), jnp.float32),
            pltpu.VMEM((ROWS_PER_TILE,), jnp.float32),
            pltpu.VMEM((128, DH), jnp.float32),
            pltpu.SemaphoreType.DMA,
            pltpu.SemaphoreType.DMA,
            pltpu.VMEM_SHARED((N_PAD, DH), jnp.float32),
            pltpu.VMEM_SHARED((N_PAD,), jnp.float32),
        ),
    )(xl, xr, src, dst)


BM = 2048


def _tc_body(x_ref, s_ref, deg_ref, ws_ref, wn_ref, b_ref, o_ref):
    inv = 1.0 / jnp.maximum(deg_ref[0, :] + deg_ref[1, :], 1.0)
    al = s_ref[0] * inv[:, None]
    ar = s_ref[1] * inv[:, None]
    o_ref[...] = (
        jnp.dot(x_ref[...], ws_ref[...], preferred_element_type=jnp.float32)
        + jnp.dot(al, wn_ref[0], preferred_element_type=jnp.float32)
        + jnp.dot(ar, wn_ref[1], preferred_element_type=jnp.float32)
        + b_ref[...]
    )


@jax.jit
def _tc_combine(x, s_agg, deg_part, w_self, w_neigh, b2d):
    grid = (N_PAD // BM,)
    return pl.pallas_call(
        _tc_body,
        grid=grid,
        in_specs=[
            pl.BlockSpec((BM, D), lambda i: (i, 0)),
            pl.BlockSpec((2, BM, DH), lambda i: (0, i, 0)),
            pl.BlockSpec((2, BM), lambda i: (0, i)),
            pl.BlockSpec((D, D), lambda i: (0, 0)),
            pl.BlockSpec((2, DH, D), lambda i: (0, 0, 0)),
            pl.BlockSpec((1, D), lambda i: (0, 0)),
        ],
        out_specs=pl.BlockSpec((BM, D), lambda i: (i, 0)),
        out_shape=jax.ShapeDtypeStruct((N_NODES, D), jnp.float32),
    )(x, s_agg, deg_part, w_self, w_neigh, b2d)


def kernel(vert_features, edges, W_self, W_neigh, b):
    x = vert_features
    xl = x[:, :DH]
    xr = x[:, DH:]
    src = edges[0].astype(jnp.int32).reshape(NT, NCHUNK, CHUNK)
    dst = edges[1].astype(jnp.int32).reshape(NT, NCHUNK, CHUNK)
    s_agg, deg_part = _sc_scatter(xl, xr, src, dst)
    wn2 = W_neigh.reshape(2, DH, D)
    return _tc_combine(x, s_agg, deg_part, W_self, wn2, b.reshape(1, D))


# speedup vs baseline: 11.4787x; 1.2477x over previous
import jax
import jax.numpy as jnp
from jax import lax
from jax.experimental import pallas as pl
from jax.experimental.pallas import tpu as pltpu
from jax.experimental.pallas import tpu_sc as plsc

N_NODES = 10000
N_PAD = 10240
D = 128
DH = D // 2
N_EDGES = 320000
NT = 16
CHUNK = 125
NCHUNK = N_EDGES // NT // CHUNK
ROWS_PER_TILE = N_PAD // NT


def _sc_body(xl_hbm, xr_hbm, src_hbm, dst_hbm, s_out, deg_out,
             src_v, dst_v, rows0_v, rows1_v, ones_v, zdeg_v, zrow_v,
             gsem0, gsem1, dsem, agg_sh, deg_sh):
    c = lax.axis_index("c")
    s = lax.axis_index("s")

    @pl.loop(0, 128)
    def _(r):
        for j in range(DH // 16):
            zrow_v[r, pl.ds(j * 16, 16)] = jnp.zeros((16,), jnp.float32)

    @pl.loop(0, ROWS_PER_TILE // 16)
    def _(k):
        zdeg_v[pl.ds(k * 16, 16)] = jnp.zeros((16,), jnp.float32)

    for k in range(8):
        ones_v[pl.ds(k * 16, 16)] = jnp.ones((16,), jnp.float32)

    for k in range(ROWS_PER_TILE // 128):
        pltpu.sync_copy(zrow_v, agg_sh.at[pl.ds(s * ROWS_PER_TILE + k * 128, 128)])
    pltpu.sync_copy(zdeg_v, deg_sh.at[pl.ds(s * ROWS_PER_TILE, ROWS_PER_TILE)])
    plsc.subcore_barrier()

    pltpu.sync_copy(src_hbm.at[s], src_v)
    pltpu.sync_copy(dst_hbm.at[s], dst_v)

    def main_loop(x_hbm, deg_first_half):
        pltpu.async_copy(x_hbm.at[src_v.at[0]], rows0_v, gsem0)

        @pl.loop(0, NCHUNK // 2)
        def _(p):
            c0 = 2 * p
            dcond = (p < NCHUNK // 4) if deg_first_half else (p >= NCHUNK // 4)
            g1 = pltpu.async_copy(x_hbm.at[src_v.at[c0 + 1]], rows1_v, gsem1)
            pltpu.make_async_copy(x_hbm.at[src_v.at[c0]], rows0_v, gsem0).wait()

            @pl.when(dcond)
            def _():
                d0 = pltpu.async_copy(ones_v.at[pl.ds(0, CHUNK)],
                                      deg_sh.at[dst_v.at[c0]], dsem, add=True)
                pltpu.sync_copy(rows0_v, agg_sh.at[dst_v.at[c0]], add=True)
                d0.wait()

            @pl.when(jnp.logical_not(dcond))
            def _():
                pltpu.sync_copy(rows0_v, agg_sh.at[dst_v.at[c0]], add=True)

            @pl.when(p < NCHUNK // 2 - 1)
            def _():
                pltpu.async_copy(x_hbm.at[src_v.at[c0 + 2]], rows0_v, gsem0)

            g1.wait()

            @pl.when(dcond)
            def _():
                d1 = pltpu.async_copy(ones_v.at[pl.ds(0, CHUNK)],
                                      deg_sh.at[dst_v.at[c0 + 1]], dsem, add=True)
                pltpu.sync_copy(rows1_v, agg_sh.at[dst_v.at[c0 + 1]], add=True)
                d1.wait()

            @pl.when(jnp.logical_not(dcond))
            def _():
                pltpu.sync_copy(rows1_v, agg_sh.at[dst_v.at[c0 + 1]], add=True)

    @pl.when(c == 0)
    def _():
        main_loop(xl_hbm, True)

    @pl.when(c == 1)
    def _():
        main_loop(xr_hbm, False)

    plsc.subcore_barrier()

    pltpu.sync_copy(agg_sh.at[pl.ds(s * ROWS_PER_TILE, ROWS_PER_TILE)],
                    s_out.at[c, pl.ds(s * ROWS_PER_TILE, ROWS_PER_TILE)])
    pltpu.sync_copy(deg_sh.at[pl.ds(s * ROWS_PER_TILE, ROWS_PER_TILE)],
                    deg_out.at[c, pl.ds(s * ROWS_PER_TILE, ROWS_PER_TILE)])


@jax.jit
def _sc_scatter(xl, xr, src, dst):
    mesh = plsc.VectorSubcoreMesh(core_axis_name="c", subcore_axis_name="s")
    return pl.kernel(
        _sc_body,
        out_type=(
            jax.ShapeDtypeStruct((2, N_PAD, DH), jnp.float32),
            jax.ShapeDtypeStruct((2, N_PAD), jnp.float32),
        ),
        mesh=mesh,
        compiler_params=pltpu.CompilerParams(use_tc_tiling_on_sc=False),
        scratch_types=(
            pltpu.VMEM((NCHUNK, CHUNK), jnp.int32),
            pltpu.VMEM((NCHUNK, CHUNK), jnp.int32),
            pltpu.VMEM((CHUNK, DH), jnp.float32),
            pltpu.VMEM((CHUNK, DH), jnp.float32),
            pltpu.VMEM((128,), jnp.float32),
            pltpu.VMEM((ROWS_PER_TILE,), jnp.float32),
            pltpu.VMEM((128, DH), jnp.float32),
            pltpu.SemaphoreType.DMA,
            pltpu.SemaphoreType.DMA,
            pltpu.SemaphoreType.DMA,
            pltpu.VMEM_SHARED((N_PAD, DH), jnp.float32),
            pltpu.VMEM_SHARED((N_PAD,), jnp.float32),
        ),
    )(xl, xr, src, dst)


BM = 2048


def _tc_body(x_ref, s_ref, deg_ref, ws_ref, wn_ref, b_ref, o_ref):
    inv = 1.0 / jnp.maximum(deg_ref[0, :] + deg_ref[1, :], 1.0)
    al = s_ref[0] * inv[:, None]
    ar = s_ref[1] * inv[:, None]
    o_ref[...] = (
        jnp.dot(x_ref[...], ws_ref[...], preferred_element_type=jnp.float32)
        + jnp.dot(al, wn_ref[0], preferred_element_type=jnp.float32)
        + jnp.dot(ar, wn_ref[1], preferred_element_type=jnp.float32)
        + b_ref[...]
    )


@jax.jit
def _tc_combine(x, s_agg, deg_part, w_self, w_neigh, b2d):
    grid = (N_PAD // BM,)
    return pl.pallas_call(
        _tc_body,
        grid=grid,
        in_specs=[
            pl.BlockSpec((BM, D), lambda i: (i, 0)),
            pl.BlockSpec((2, BM, DH), lambda i: (0, i, 0)),
            pl.BlockSpec((2, BM), lambda i: (0, i)),
            pl.BlockSpec((D, D), lambda i: (0, 0)),
            pl.BlockSpec((2, DH, D), lambda i: (0, 0, 0)),
            pl.BlockSpec((1, D), lambda i: (0, 0)),
        ],
        out_specs=pl.BlockSpec((BM, D), lambda i: (i, 0)),
        out_shape=jax.ShapeDtypeStruct((N_NODES, D), jnp.float32),
    )(x, s_agg, deg_part, w_self, w_neigh, b2d)


def kernel(vert_features, edges, W_self, W_neigh, b):
    x = vert_features
    xl = x[:, :DH]
    xr = x[:, DH:]
    src = edges[0].astype(jnp.int32).reshape(NT, NCHUNK, CHUNK)
    dst = edges[1].astype(jnp.int32).reshape(NT, NCHUNK, CHUNK)
    s_agg, deg_part = _sc_scatter(xl, xr, src, dst)
    wn2 = W_neigh.reshape(2, DH, D)
    return _tc_combine(x, s_agg, deg_part, W_self, wn2, b.reshape(1, D))


# speedup vs baseline: 14.2140x; 1.2383x over previous
import jax
import jax.numpy as jnp
from jax import lax
from jax.experimental import pallas as pl
from jax.experimental.pallas import tpu as pltpu
from jax.experimental.pallas import tpu_sc as plsc

N_NODES = 10000
N_PAD = 10240
D = 128
DH = D // 2
N_EDGES = 320000
NT = 16
CHUNK = 125
NCHUNK = N_EDGES // NT // CHUNK
ROWS_PER_TILE = N_PAD // NT


def _sc_body(xl_hbm, xr_hbm, src_hbm, dst_hbm, s_out, deg_out,
             src_v, dst_v, rows0_v, rows1_v, rows2_v, rows3_v, ones_v,
             zdeg_v, zrow_v, gsem0, gsem1, gsem2, gsem3, dsem,
             agg_sh, deg_sh):
    c = lax.axis_index("c")
    s = lax.axis_index("s")

    @pl.loop(0, 128)
    def _(r):
        for j in range(DH // 16):
            zrow_v[r, pl.ds(j * 16, 16)] = jnp.zeros((16,), jnp.float32)

    @pl.loop(0, ROWS_PER_TILE // 16)
    def _(k):
        zdeg_v[pl.ds(k * 16, 16)] = jnp.zeros((16,), jnp.float32)

    for k in range(8):
        ones_v[pl.ds(k * 16, 16)] = jnp.ones((16,), jnp.float32)

    for k in range(ROWS_PER_TILE // 128):
        pltpu.sync_copy(zrow_v, agg_sh.at[pl.ds(s * ROWS_PER_TILE + k * 128, 128)])
    pltpu.sync_copy(zdeg_v, deg_sh.at[pl.ds(s * ROWS_PER_TILE, ROWS_PER_TILE)])
    plsc.subcore_barrier()

    pltpu.sync_copy(src_hbm.at[s], src_v)
    pltpu.sync_copy(dst_hbm.at[s], dst_v)

    def main_loop(x_hbm, deg_first_half):
        rows = (rows0_v, rows1_v, rows2_v, rows3_v)
        gsems = (gsem0, gsem1, gsem2, gsem3)
        for i in range(4):
            pltpu.async_copy(x_hbm.at[src_v.at[i]], rows[i], gsems[i])

        @pl.loop(0, NCHUNK // 4)
        def _(p):
            c0 = 4 * p
            dcond = (p < NCHUNK // 8) if deg_first_half else (p >= NCHUNK // 8)
            for i in range(4):
                pltpu.make_async_copy(x_hbm.at[src_v.at[c0 + i]],
                                      rows[i], gsems[i]).wait()

                @pl.when(dcond)
                def _(i=i):
                    d = pltpu.async_copy(ones_v.at[pl.ds(0, CHUNK)],
                                         deg_sh.at[dst_v.at[c0 + i]], dsem,
                                         add=True)
                    pltpu.sync_copy(rows[i], agg_sh.at[dst_v.at[c0 + i]],
                                    add=True)
                    d.wait()

                @pl.when(jnp.logical_not(dcond))
                def _(i=i):
                    pltpu.sync_copy(rows[i], agg_sh.at[dst_v.at[c0 + i]],
                                    add=True)

                @pl.when(p < NCHUNK // 4 - 1)
                def _(i=i):
                    pltpu.async_copy(x_hbm.at[src_v.at[c0 + i + 4]],
                                     rows[i], gsems[i])

    @pl.when(c == 0)
    def _():
        main_loop(xl_hbm, True)

    @pl.when(c == 1)
    def _():
        main_loop(xr_hbm, False)

    plsc.subcore_barrier()

    pltpu.sync_copy(agg_sh.at[pl.ds(s * ROWS_PER_TILE, ROWS_PER_TILE)],
                    s_out.at[c, pl.ds(s * ROWS_PER_TILE, ROWS_PER_TILE)])
    pltpu.sync_copy(deg_sh.at[pl.ds(s * ROWS_PER_TILE, ROWS_PER_TILE)],
                    deg_out.at[c, pl.ds(s * ROWS_PER_TILE, ROWS_PER_TILE)])


@jax.jit
def _sc_scatter(xl, xr, src, dst):
    mesh = plsc.VectorSubcoreMesh(core_axis_name="c", subcore_axis_name="s")
    return pl.kernel(
        _sc_body,
        out_type=(
            jax.ShapeDtypeStruct((2, N_PAD, DH), jnp.float32),
            jax.ShapeDtypeStruct((2, N_PAD), jnp.float32),
        ),
        mesh=mesh,
        compiler_params=pltpu.CompilerParams(use_tc_tiling_on_sc=False),
        scratch_types=(
            pltpu.VMEM((NCHUNK, CHUNK), jnp.int32),
            pltpu.VMEM((NCHUNK, CHUNK), jnp.int32),
            pltpu.VMEM((CHUNK, DH), jnp.float32),
            pltpu.VMEM((CHUNK, DH), jnp.float32),
            pltpu.VMEM((CHUNK, DH), jnp.float32),
            pltpu.VMEM((CHUNK, DH), jnp.float32),
            pltpu.VMEM((128,), jnp.float32),
            pltpu.VMEM((ROWS_PER_TILE,), jnp.float32),
            pltpu.VMEM((128, DH), jnp.float32),
            pltpu.SemaphoreType.DMA,
            pltpu.SemaphoreType.DMA,
            pltpu.SemaphoreType.DMA,
            pltpu.SemaphoreType.DMA,
            pltpu.SemaphoreType.DMA,
            pltpu.VMEM_SHARED((N_PAD, DH), jnp.float32),
            pltpu.VMEM_SHARED((N_PAD,), jnp.float32),
        ),
    )(xl, xr, src, dst)


BM = 2048


def _tc_pre_body(x_ref, ws_ref, b_ref, z_ref):
    z_ref[...] = (
        jnp.dot(x_ref[...], ws_ref[...], preferred_element_type=jnp.float32)
        + b_ref[...]
    )


@jax.jit
def _tc_pre(x, w_self, b2d):
    return pl.pallas_call(
        _tc_pre_body,
        grid=(N_PAD // BM,),
        in_specs=[
            pl.BlockSpec((BM, D), lambda i: (i, 0)),
            pl.BlockSpec((D, D), lambda i: (0, 0)),
            pl.BlockSpec((1, D), lambda i: (0, 0)),
        ],
        out_specs=pl.BlockSpec((BM, D), lambda i: (i, 0)),
        out_shape=jax.ShapeDtypeStruct((N_NODES, D), jnp.float32),
    )(x, w_self, b2d)


def _tc_post_body(z_ref, s_ref, deg_ref, wn_ref, o_ref):
    inv = 1.0 / jnp.maximum(deg_ref[0, :] + deg_ref[1, :], 1.0)
    al = s_ref[0] * inv[:, None]
    ar = s_ref[1] * inv[:, None]
    o_ref[...] = (
        z_ref[...]
        + jnp.dot(al, wn_ref[0], preferred_element_type=jnp.float32)
        + jnp.dot(ar, wn_ref[1], preferred_element_type=jnp.float32)
    )


@jax.jit
def _tc_post(z, s_agg, deg_part, w_neigh2):
    return pl.pallas_call(
        _tc_post_body,
        grid=(N_PAD // BM,),
        in_specs=[
            pl.BlockSpec((BM, D), lambda i: (i, 0)),
            pl.BlockSpec((2, BM, DH), lambda i: (0, i, 0)),
            pl.BlockSpec((2, BM), lambda i: (0, i)),
            pl.BlockSpec((2, DH, D), lambda i: (0, 0, 0)),
        ],
        out_specs=pl.BlockSpec((BM, D), lambda i: (i, 0)),
        out_shape=jax.ShapeDtypeStruct((N_NODES, D), jnp.float32),
    )(z, s_agg, deg_part, w_neigh2)


def kernel(vert_features, edges, W_self, W_neigh, b):
    x = vert_features
    xl = x[:, :DH]
    xr = x[:, DH:]
    src = edges[0].astype(jnp.int32).reshape(NT, NCHUNK, CHUNK)
    dst = edges[1].astype(jnp.int32).reshape(NT, NCHUNK, CHUNK)
    z = _tc_pre(x, W_self, b.reshape(1, D))
    s_agg, deg_part = _sc_scatter(xl, xr, src, dst)
    wn2 = W_neigh.reshape(2, DH, D)
    return _tc_post(z, s_agg, deg_part, wn2)
